# Initial kernel scaffold; baseline (speedup 1.0000x reference)
import jax
import jax.numpy as jnp
from jax.experimental import pallas as pl


def kernel(imgbatch, graph_feats, edge_index, edge_weights, conv_k, conv_b, W1, b1, W2, b2, W3a, b3a, W3b, b3b):
    raise NotImplementedError("write your pallas kernel here")


# speedup vs baseline: 12.1149x; 12.1149x over previous
import functools

import jax
import jax.numpy as jnp
from jax import lax
from jax.experimental import pallas as pl
from jax.experimental.pallas import tpu as pltpu
from jax.experimental.pallas import tpu_sc as plsc

N = 10000
E = 320000
NP = 10240
EF = E + N
K = 320
EP = 337920
NC, NS, L = 2, 16, 16

@functools.cache
def _mesh():
  return plsc.VectorSubcoreMesh(
      core_axis_name="c", subcore_axis_name="s", num_cores=NC, num_subcores=NS)


def _deg_body(dst_hbm, ew_hbm, out_hbm, dst_v, ew_v, acc, sem):
  c = lax.axis_index("c")
  s = lax.axis_index("s")
  w = c * NS + s

  def _z(j, _):
    ew_v[pl.ds(j * L, L)] = jnp.zeros((L,), jnp.float32)
    return 0
  lax.fori_loop(0, K // L, _z, 0)
  pltpu.sync_copy(ew_v, acc.at[pl.ds(s * 640, K)])
  pltpu.sync_copy(ew_v, acc.at[pl.ds(s * 640 + K, K)])
  plsc.subcore_barrier()

  et = EP // (NC * NS)
  nwin = et // K

  def _win(j, _):
    base = w * et + j * K
    pltpu.sync_copy(dst_hbm.at[pl.ds(base, K)], dst_v)
    pltpu.sync_copy(ew_hbm.at[pl.ds(base, K)], ew_v)
    pltpu.sync_copy(ew_v, acc.at[dst_v], add=True)
    return 0
  lax.fori_loop(0, nwin, _win, 0)

  plsc.subcore_barrier()
  pltpu.sync_copy(acc.at[pl.ds(s * 640, 640)],
                  out_hbm.at[c, pl.ds(s * 640, 640)])


@functools.cache
def _deg_call():
  return pl.kernel(
      _deg_body,
      out_type=jax.ShapeDtypeStruct((NC, NP), jnp.float32),
      mesh=_mesh(),
      scratch_types=[
          pltpu.VMEM((K,), jnp.int32),
          pltpu.VMEM((K,), jnp.float32),
          pltpu.VMEM_SHARED((NP,), jnp.float32),
          pltpu.SemaphoreType.DMA,
      ],
  )


def _nrm_body(src_hbm, dst_hbm, ew_hbm, dinv_hbm, out_hbm,
              src_v, dst_v, ew_v, a_v, b_v, sem):
  c = lax.axis_index("c")
  s = lax.axis_index("s")
  w = c * NS + s

  et = EP // (NC * NS)
  nwin = et // K

  def _win(j, _):
    base = w * et + j * K
    pltpu.sync_copy(src_hbm.at[pl.ds(base, K)], src_v)
    pltpu.sync_copy(dst_hbm.at[pl.ds(base, K)], dst_v)
    pltpu.sync_copy(ew_hbm.at[pl.ds(base, K)], ew_v)
    cp_a = pltpu.async_copy(dinv_hbm.at[src_v], a_v, sem)
    cp_b = pltpu.async_copy(dinv_hbm.at[dst_v], b_v, sem)
    cp_a.wait()
    cp_b.wait()

    def _m(t, _):
      sl = pl.ds(t * L, L)
      ew_v[sl] = a_v[sl] * ew_v[sl] * b_v[sl]
      return 0
    lax.fori_loop(0, K // L, _m, 0)
    pltpu.sync_copy(ew_v, out_hbm.at[pl.ds(base, K)])
    return 0
  lax.fori_loop(0, nwin, _win, 0)


@functools.cache
def _nrm_call():
  return pl.kernel(
      _nrm_body,
      out_type=jax.ShapeDtypeStruct((EP,), jnp.float32),
      mesh=_mesh(),
      scratch_types=[
          pltpu.VMEM((K,), jnp.int32),
          pltpu.VMEM((K,), jnp.int32),
          pltpu.VMEM((K,), jnp.float32),
          pltpu.VMEM((K,), jnp.float32),
          pltpu.VMEM((K,), jnp.float32),
          pltpu.SemaphoreType.DMA,
      ],
  )


DC = 128


def _ax_body(split_edges, x_hbm, src_hbm, dst_hbm, nrm_hbm, out_hbm,
             src_v, dst_v, nrm_v, rows, acc, sem):
  c = lax.axis_index("c")
  s = lax.axis_index("s")

  def _z(r, _):
    for f in range(DC // L):
      rows[r, pl.ds(f * L, L)] = jnp.zeros((L,), jnp.float32)
    return 0
  lax.fori_loop(0, K, _z, 0)
  pltpu.sync_copy(rows, acc.at[pl.ds(s * 640, K)])
  pltpu.sync_copy(rows, acc.at[pl.ds(s * 640 + K, K)])
  plsc.subcore_barrier()

  if split_edges:
    et = EP // (NC * NS)
    first = (c * NS + s) * et
  else:
    et = EP // NS
    first = s * et
  nwin = et // K

  def _win(j, _):
    base = first + j * K
    pltpu.sync_copy(src_hbm.at[pl.ds(base, K)], src_v)
    pltpu.sync_copy(dst_hbm.at[pl.ds(base, K)], dst_v)
    pltpu.sync_copy(nrm_hbm.at[pl.ds(base, K)], nrm_v)

    if not split_edges:
      coff = c * NP

      def _off(t, _):
        sl = pl.ds(t * L, L)
        src_v[sl] = src_v[sl] + coff
        return 0
      lax.fori_loop(0, K // L, _off, 0)

    pltpu.async_copy(x_hbm.at[src_v], rows, sem).wait()

    def _mul(t, _):
      nv = nrm_v[pl.ds(t * L, L)]
      for i in range(L):
        e = t * L + i
        ns = nv[i]
        for f in range(DC // L):
          sl = pl.ds(f * L, L)
          rows[e, sl] = rows[e, sl] * ns
      return 0
    lax.fori_loop(0, K // L, _mul, 0)

    pltpu.sync_copy(rows, acc.at[dst_v], add=True)
    return 0
  lax.fori_loop(0, nwin, _win, 0)

  plsc.subcore_barrier()
  pltpu.sync_copy(acc.at[pl.ds(s * 640, 640)],
                  out_hbm.at[c, pl.ds(s * 640, 640)])


@functools.cache
def _make_ax_call(split_edges):
  return pl.kernel(
      functools.partial(_ax_body, split_edges),
      out_type=jax.ShapeDtypeStruct((NC, NP, DC), jnp.float32),
      mesh=_mesh(),
      scratch_types=[
          pltpu.VMEM((K,), jnp.int32),
          pltpu.VMEM((K,), jnp.int32),
          pltpu.VMEM((K,), jnp.float32),
          pltpu.VMEM((K, DC), jnp.float32),
          pltpu.VMEM_SHARED((NP, DC), jnp.float32),
          pltpu.SemaphoreType.DMA,
      ],
  )


BN_CONV = 200


def _conv_body(img_ref, kw_ref, kb_ref, out_ref):
  x = img_ref[...]
  col = lax.broadcasted_iota(jnp.int32, (1, 2048), 1)
  wm = col % 32
  hm = (col % 1024) // 32

  shifted = []
  for dy in range(3):
    for dx in range(3):
      o = (dy - 1) * 32 + (dx - 1)
      v = jnp.roll(x, -o, axis=1) if o != 0 else x
      ok = ((hm + (dy - 1) >= 0) & (hm + (dy - 1) < 32)
            & (wm + (dx - 1) >= 0) & (wm + (dx - 1) < 32))
      shifted.append(jnp.where(ok, v, 0.0))

  feats = []
  for co in range(6):
    acc = jnp.zeros((BN_CONV, 1024), jnp.float32)
    for ci in range(2):
      half = slice(ci * 1024, (ci + 1) * 1024)
      for t in range(9):
        acc = acc + shifted[t][:, half] * kw_ref[co * 18 + ci * 9 + t]
    m = jnp.max(acc, axis=1, keepdims=True) + kb_ref[co]
    feats.append(jnp.maximum(m, 0.0))
  feats.append(jnp.zeros((BN_CONV, 2), jnp.float32))
  out_ref[...] = jnp.concatenate(feats, axis=1)


def _conv_call(imgflat, kw, kb):
  return pl.pallas_call(
      _conv_body,
      grid=(N // BN_CONV,),
      in_specs=[
          pl.BlockSpec((BN_CONV, 2048), lambda i: (i, 0)),
          pl.BlockSpec(memory_space=pltpu.SMEM),
          pl.BlockSpec(memory_space=pltpu.SMEM),
      ],
      out_specs=pl.BlockSpec((BN_CONV, 8), lambda i: (i, 0)),
      out_shape=jax.ShapeDtypeStruct((N, 8), jnp.float32),
  )(imgflat, kw, kb)


def _dinv_body(degp_ref, out_ref):
  d = jnp.sum(degp_ref[...], axis=0, keepdims=True)
  out_ref[...] = jnp.where(d > 0, lax.rsqrt(d), 0.0)


def _dinv_call(degp):
  return pl.pallas_call(
      _dinv_body,
      out_shape=jax.ShapeDtypeStruct((1, NP), jnp.float32),
  )(degp)


BN_MM = 1024


def _mm1_body(x0_ref, x1_ref, w_ref, b_ref, out_ref):
  x = x0_ref[...] + x1_ref[...]
  h = jnp.dot(x, w_ref[...], preferred_element_type=jnp.float32) + b_ref[...]
  out_ref[...] = jnp.maximum(h, 0.0)


def _mm1_call(x0, x1, w1, b1):
  return pl.pallas_call(
      _mm1_body,
      grid=(NP // BN_MM,),
      in_specs=[
          pl.BlockSpec((BN_MM, 128), lambda i: (i, 0)),
          pl.BlockSpec((BN_MM, 128), lambda i: (i, 0)),
          pl.BlockSpec((128, 256), lambda i: (0, 0)),
          pl.BlockSpec((1, 256), lambda i: (0, 0)),
      ],
      out_specs=pl.BlockSpec((BN_MM, 256), lambda i: (i, 0)),
      out_shape=jax.ShapeDtypeStruct((NP, 256), jnp.float32),
  )(x0, x1, w1, b1)


def _mm2q_body(x0_ref, x1_ref, w_ref, b_ref, w3_ref, out_ref):
  h = (jnp.dot(x0_ref[...], w_ref[0:128, :], preferred_element_type=jnp.float32)
       + jnp.dot(x1_ref[...], w_ref[128:256, :],
                 preferred_element_type=jnp.float32)
       + b_ref[...])
  h = jnp.maximum(h, 0.0)
  out_ref[...] = jnp.dot(h, w3_ref[...], preferred_element_type=jnp.float32)


def _mm2q_call(x0, x1, w2, b2, w3):
  return pl.pallas_call(
      _mm2q_body,
      grid=(NP // BN_MM,),
      in_specs=[
          pl.BlockSpec((BN_MM, 128), lambda i: (i, 0)),
          pl.BlockSpec((BN_MM, 128), lambda i: (i, 0)),
          pl.BlockSpec((256, 512), lambda i: (0, 0)),
          pl.BlockSpec((1, 512), lambda i: (0, 0)),
          pl.BlockSpec((512, 128), lambda i: (0, 0)),
      ],
      out_specs=pl.BlockSpec((BN_MM, 128), lambda i: (i, 0)),
      out_shape=jax.ShapeDtypeStruct((NP, 128), jnp.float32),
  )(x0, x1, w2, b2, w3)


def _bias_body(p0_ref, p1_ref, b_ref, out_ref):
  out_ref[...] = p0_ref[...] + p1_ref[...] + b_ref[...]


def _bias_call(p0, p1, b3):
  return pl.pallas_call(
      _bias_body,
      grid=(NP // BN_MM,),
      in_specs=[
          pl.BlockSpec((BN_MM, DC), lambda i: (i, 0)),
          pl.BlockSpec((BN_MM, DC), lambda i: (i, 0)),
          pl.BlockSpec((1, DC), lambda i: (0, 0)),
      ],
      out_specs=pl.BlockSpec((BN_MM, DC), lambda i: (i, 0)),
      out_shape=jax.ShapeDtypeStruct((NP, DC), jnp.float32),
  )(p0, p1, b3)


def kernel(imgbatch, graph_feats, edge_index, edge_weights, conv_k, conv_b,
           W1, b1, W2, b2, W3a, b3a, W3b, b3b):
  src = edge_index[0].astype(jnp.int32)
  dst = edge_index[1].astype(jnp.int32)
  loop = jnp.arange(N, dtype=jnp.int32)
  npad = EP - EF
  pad_idx = jnp.arange(npad, dtype=jnp.int32) % N
  src_f = jnp.concatenate([src, loop, pad_idx])
  dst_f = jnp.concatenate([dst, loop, pad_idx])
  ew_f = jnp.concatenate([edge_weights.astype(jnp.float32),
                          jnp.ones((N,), jnp.float32),
                          jnp.zeros((npad,), jnp.float32)])

  imgflat = imgbatch.reshape(N, 2048)
  kw = conv_k.reshape(6 * 18)
  kb = conv_b.reshape(6)

  degp = _deg_call()(dst_f, ew_f)
  dinv = _dinv_call(degp).reshape(NP)
  nrm = _nrm_call()(src_f, dst_f, ew_f, dinv)

  imgf = _conv_call(imgflat, kw, kb)
  z = jnp.concatenate([imgf[:, :6], graph_feats], axis=1)
  z = jnp.pad(z, ((0, NP - N), (0, 0)))

  p1 = _make_ax_call(True)(z, src_f, dst_f, nrm)
  h1 = _mm1_call(p1[0], p1[1], W1, b1.reshape(1, 256))

  xcat2 = jnp.concatenate([h1[:, :128], h1[:, 128:]], axis=0)
  p2 = _make_ax_call(False)(xcat2, src_f, dst_f, nrm)
  w3 = jnp.concatenate([W3a, W3b], axis=1)
  w3 = jnp.pad(w3, ((0, 0), (0, 126)))
  q = _mm2q_call(p2[0], p2[1], W2, b2.reshape(1, 512), w3)

  p3 = _make_ax_call(True)(q, src_f, dst_f, nrm)
  b3 = jnp.concatenate([b3a, b3b]).reshape(1, 2)
  b3 = jnp.pad(b3, ((0, 0), (0, DC - 2)))
  out = _bias_call(p3[0], p3[1], b3)

  mu = out[:N, 0].reshape(N, 1)
  log_var = out[:N, 1].reshape(N, 1)
  return (mu, log_var)


# speedup vs baseline: 12.9222x; 1.0666x over previous
import functools

import jax
import jax.numpy as jnp
from jax import lax
from jax.experimental import pallas as pl
from jax.experimental.pallas import tpu as pltpu
from jax.experimental.pallas import tpu_sc as plsc

N = 10000
E = 320000
NP = 10240
EF = E + N
K = 320
EP = 337920
NC, NS, L = 2, 16, 16

@functools.cache
def _mesh():
  return plsc.VectorSubcoreMesh(
      core_axis_name="c", subcore_axis_name="s", num_cores=NC, num_subcores=NS)


def _deg_body(dst_hbm, ew_hbm, out_hbm, dst_v, ew_v, acc, sem):
  c = lax.axis_index("c")
  s = lax.axis_index("s")
  w = c * NS + s

  def _z(j, _):
    ew_v[pl.ds(j * L, L)] = jnp.zeros((L,), jnp.float32)
    return 0
  lax.fori_loop(0, K // L, _z, 0)
  pltpu.sync_copy(ew_v, acc.at[pl.ds(s * 640, K)])
  pltpu.sync_copy(ew_v, acc.at[pl.ds(s * 640 + K, K)])
  plsc.subcore_barrier()

  et = EP // (NC * NS)
  nwin = et // K

  def _win(j, _):
    base = w * et + j * K
    pltpu.sync_copy(dst_hbm.at[pl.ds(base, K)], dst_v)
    pltpu.sync_copy(ew_hbm.at[pl.ds(base, K)], ew_v)
    pltpu.sync_copy(ew_v, acc.at[dst_v], add=True)
    return 0
  lax.fori_loop(0, nwin, _win, 0)

  plsc.subcore_barrier()
  pltpu.sync_copy(acc.at[pl.ds(s * 640, 640)],
                  out_hbm.at[c, pl.ds(s * 640, 640)])


@functools.cache
def _deg_call():
  return pl.kernel(
      _deg_body,
      out_type=jax.ShapeDtypeStruct((NC, NP), jnp.float32),
      mesh=_mesh(),
      scratch_types=[
          pltpu.VMEM((K,), jnp.int32),
          pltpu.VMEM((K,), jnp.float32),
          pltpu.VMEM_SHARED((NP,), jnp.float32),
          pltpu.SemaphoreType.DMA,
      ],
  )


def _axq_body(x_hbm, src_hbm, dst_hbm, ew_hbm, out_hbm,
              src_v, dst_v, ew_v, vals_v, acc, sem):
  c = lax.axis_index("c")
  s = lax.axis_index("s")
  coff = c * NP

  def _z(j, _):
    ew_v[pl.ds(j * L, L)] = jnp.zeros((L,), jnp.float32)
    return 0
  lax.fori_loop(0, K // L, _z, 0)
  pltpu.sync_copy(ew_v, acc.at[pl.ds(s * 640, K)])
  pltpu.sync_copy(ew_v, acc.at[pl.ds(s * 640 + K, K)])
  plsc.subcore_barrier()

  et = EP // NS
  nwin = et // K

  def _win(j, _):
    base = s * et + j * K
    pltpu.sync_copy(src_hbm.at[pl.ds(base, K)], src_v)
    pltpu.sync_copy(dst_hbm.at[pl.ds(base, K)], dst_v)
    pltpu.sync_copy(ew_hbm.at[pl.ds(base, K)], ew_v)

    def _off(t, _):
      sl = pl.ds(t * L, L)
      src_v[sl] = src_v[sl] + coff
      return 0
    lax.fori_loop(0, K // L, _off, 0)

    pltpu.async_copy(x_hbm.at[src_v], vals_v, sem).wait()

    def _mul(t, _):
      sl = pl.ds(t * L, L)
      vals_v[sl] = vals_v[sl] * ew_v[sl]
      return 0
    lax.fori_loop(0, K // L, _mul, 0)

    pltpu.sync_copy(vals_v, acc.at[dst_v], add=True)
    return 0
  lax.fori_loop(0, nwin, _win, 0)

  plsc.subcore_barrier()
  pltpu.sync_copy(acc.at[pl.ds(s * 640, 640)],
                  out_hbm.at[c, pl.ds(s * 640, 640)])


@functools.cache
def _axq_call():
  return pl.kernel(
      _axq_body,
      out_type=jax.ShapeDtypeStruct((NC, NP), jnp.float32),
      mesh=_mesh(),
      scratch_types=[
          pltpu.VMEM((K,), jnp.int32),
          pltpu.VMEM((K,), jnp.int32),
          pltpu.VMEM((K,), jnp.float32),
          pltpu.VMEM((K,), jnp.float32),
          pltpu.VMEM_SHARED((NP,), jnp.float32),
          pltpu.SemaphoreType.DMA,
      ],
  )


DC = 128


def _ax_body(split_edges, x_hbm, src_hbm, dst_hbm, nrm_hbm, out_hbm,
             src_v, dst_v, nrm_v, rows, acc, sem):
  c = lax.axis_index("c")
  s = lax.axis_index("s")

  def _z(r, _):
    for f in range(DC // L):
      rows[r, pl.ds(f * L, L)] = jnp.zeros((L,), jnp.float32)
    return 0
  lax.fori_loop(0, K, _z, 0)
  pltpu.sync_copy(rows, acc.at[pl.ds(s * 640, K)])
  pltpu.sync_copy(rows, acc.at[pl.ds(s * 640 + K, K)])
  plsc.subcore_barrier()

  if split_edges:
    et = EP // (NC * NS)
    first = (c * NS + s) * et
  else:
    et = EP // NS
    first = s * et
  nwin = et // K

  def _win(j, _):
    base = first + j * K
    pltpu.sync_copy(src_hbm.at[pl.ds(base, K)], src_v)
    pltpu.sync_copy(dst_hbm.at[pl.ds(base, K)], dst_v)
    pltpu.sync_copy(nrm_hbm.at[pl.ds(base, K)], nrm_v)

    if not split_edges:
      coff = c * NP

      def _off(t, _):
        sl = pl.ds(t * L, L)
        src_v[sl] = src_v[sl] + coff
        return 0
      lax.fori_loop(0, K // L, _off, 0)

    pltpu.async_copy(x_hbm.at[src_v], rows, sem).wait()

    def _mul(t, _):
      nv = nrm_v[pl.ds(t * L, L)]
      for i in range(L):
        e = t * L + i
        ns = nv[i]
        for f in range(DC // L):
          sl = pl.ds(f * L, L)
          rows[e, sl] = rows[e, sl] * ns
      return 0
    lax.fori_loop(0, K // L, _mul, 0)

    pltpu.sync_copy(rows, acc.at[dst_v], add=True)
    return 0
  lax.fori_loop(0, nwin, _win, 0)

  plsc.subcore_barrier()
  pltpu.sync_copy(acc.at[pl.ds(s * 640, 640)],
                  out_hbm.at[c, pl.ds(s * 640, 640)])


@functools.cache
def _make_ax_call(split_edges):
  return pl.kernel(
      functools.partial(_ax_body, split_edges),
      out_type=jax.ShapeDtypeStruct((NC, NP, DC), jnp.float32),
      mesh=_mesh(),
      scratch_types=[
          pltpu.VMEM((K,), jnp.int32),
          pltpu.VMEM((K,), jnp.int32),
          pltpu.VMEM((K,), jnp.float32),
          pltpu.VMEM((K, DC), jnp.float32),
          pltpu.VMEM_SHARED((NP, DC), jnp.float32),
          pltpu.SemaphoreType.DMA,
      ],
  )


BN_CONV = 200


def _conv_body(img_ref, gfp_ref, dv_ref, kw_ref, kb_ref, out_ref):
  x = img_ref[...]
  col = lax.broadcasted_iota(jnp.int32, (1, 2048), 1)
  wm = col % 32
  hm = (col % 1024) // 32

  shifted = []
  for dy in range(3):
    for dx in range(3):
      o = (dy - 1) * 32 + (dx - 1)
      v = jnp.roll(x, -o, axis=1) if o != 0 else x
      ok = ((hm + (dy - 1) >= 0) & (hm + (dy - 1) < 32)
            & (wm + (dx - 1) >= 0) & (wm + (dx - 1) < 32))
      shifted.append(jnp.where(ok, v, 0.0))

  feats = []
  for co in range(6):
    acc = jnp.zeros((BN_CONV, 1024), jnp.float32)
    for ci in range(2):
      half = slice(ci * 1024, (ci + 1) * 1024)
      for t in range(9):
        acc = acc + shifted[t][:, half] * kw_ref[co * 18 + ci * 9 + t]
    m = jnp.max(acc, axis=1, keepdims=True) + kb_ref[co]
    feats.append(jnp.maximum(m, 0.0))
  imgf = jnp.concatenate(feats, axis=1)
  imgf = jnp.pad(imgf, ((0, 0), (0, 122)))
  out_ref[...] = (imgf + gfp_ref[...]) * dv_ref[...]


def _conv_call(imgflat, gfp, dv, kw, kb):
  return pl.pallas_call(
      _conv_body,
      grid=(N // BN_CONV,),
      in_specs=[
          pl.BlockSpec((BN_CONV, 2048), lambda i: (i, 0)),
          pl.BlockSpec((BN_CONV, 128), lambda i: (i, 0)),
          pl.BlockSpec((BN_CONV, 1), lambda i: (i, 0)),
          pl.BlockSpec(memory_space=pltpu.SMEM),
          pl.BlockSpec(memory_space=pltpu.SMEM),
      ],
      out_specs=pl.BlockSpec((BN_CONV, 128), lambda i: (i, 0)),
      out_shape=jax.ShapeDtypeStruct((N, 128), jnp.float32),
  )(imgflat, gfp, dv, kw, kb)


def _dinv_body(degp_ref, out_ref):
  d = jnp.sum(degp_ref[...], axis=0, keepdims=True)
  out_ref[...] = jnp.where(d > 0, lax.rsqrt(d), 0.0)


def _dinv_call(degp):
  return pl.pallas_call(
      _dinv_body,
      out_shape=jax.ShapeDtypeStruct((1, NP), jnp.float32),
  )(degp)


BN_MM = 1024


def _mm1_body(x0_ref, x1_ref, dv_ref, w_ref, b_ref, out_ref):
  dv = dv_ref[...]
  x = (x0_ref[...] + x1_ref[...]) * dv
  h = jnp.dot(x, w_ref[...], preferred_element_type=jnp.float32) + b_ref[...]
  out_ref[...] = jnp.maximum(h, 0.0) * dv


def _mm1_call(x0, x1, dv, w1, b1):
  return pl.pallas_call(
      _mm1_body,
      grid=(NP // BN_MM,),
      in_specs=[
          pl.BlockSpec((BN_MM, 128), lambda i: (i, 0)),
          pl.BlockSpec((BN_MM, 128), lambda i: (i, 0)),
          pl.BlockSpec((BN_MM, 1), lambda i: (i, 0)),
          pl.BlockSpec((128, 256), lambda i: (0, 0)),
          pl.BlockSpec((1, 256), lambda i: (0, 0)),
      ],
      out_specs=pl.BlockSpec((BN_MM, 256), lambda i: (i, 0)),
      out_shape=jax.ShapeDtypeStruct((NP, 256), jnp.float32),
  )(x0, x1, dv, w1, b1)


def _mm2q_body(x0_ref, x1_ref, dv_ref, w_ref, b_ref, w3_ref, out_ref):
  dv = dv_ref[...]
  h = (jnp.dot(x0_ref[...] * dv, w_ref[0:128, :],
               preferred_element_type=jnp.float32)
       + jnp.dot(x1_ref[...] * dv, w_ref[128:256, :],
                 preferred_element_type=jnp.float32)
       + b_ref[...])
  h = jnp.maximum(h, 0.0)
  q = jnp.dot(h, w3_ref[...], preferred_element_type=jnp.float32)
  out_ref[...] = q * dv


def _mm2q_call(x0, x1, dv, w2, b2, w3):
  return pl.pallas_call(
      _mm2q_body,
      grid=(NP // BN_MM,),
      in_specs=[
          pl.BlockSpec((BN_MM, 128), lambda i: (i, 0)),
          pl.BlockSpec((BN_MM, 128), lambda i: (i, 0)),
          pl.BlockSpec((BN_MM, 1), lambda i: (i, 0)),
          pl.BlockSpec((256, 512), lambda i: (0, 0)),
          pl.BlockSpec((1, 512), lambda i: (0, 0)),
          pl.BlockSpec((512, 128), lambda i: (0, 0)),
      ],
      out_specs=pl.BlockSpec((BN_MM, 128), lambda i: (i, 0)),
      out_shape=jax.ShapeDtypeStruct((NP, 128), jnp.float32),
  )(x0, x1, dv, w2, b2, w3)


def _bias_body(p_ref, dv_ref, b_ref, out_ref):
  out_ref[...] = p_ref[...] * dv_ref[...] + b_ref[...]


def _bias_call(p3, dv, b3):
  return pl.pallas_call(
      _bias_body,
      in_specs=[
          pl.BlockSpec((2, NP), lambda: (0, 0)),
          pl.BlockSpec((1, NP), lambda: (0, 0)),
          pl.BlockSpec((2, 1), lambda: (0, 0)),
      ],
      out_specs=pl.BlockSpec((2, NP), lambda: (0, 0)),
      out_shape=jax.ShapeDtypeStruct((2, NP), jnp.float32),
  )(p3, dv, b3)


def kernel(imgbatch, graph_feats, edge_index, edge_weights, conv_k, conv_b,
           W1, b1, W2, b2, W3a, b3a, W3b, b3b):
  src = edge_index[0].astype(jnp.int32)
  dst = edge_index[1].astype(jnp.int32)
  loop = jnp.arange(N, dtype=jnp.int32)
  npad = EP - EF
  pad_idx = jnp.arange(npad, dtype=jnp.int32) % N
  src_f = jnp.concatenate([src, loop, pad_idx])
  dst_f = jnp.concatenate([dst, loop, pad_idx])
  ew_f = jnp.concatenate([edge_weights.astype(jnp.float32),
                          jnp.ones((N,), jnp.float32),
                          jnp.zeros((npad,), jnp.float32)])

  imgflat = imgbatch.reshape(N, 2048)
  kw = conv_k.reshape(6 * 18)
  kb = conv_b.reshape(6)

  degp = _deg_call()(dst_f, ew_f)
  dinv = _dinv_call(degp)
  dv_col = dinv.reshape(NP, 1)

  gfp = jnp.pad(graph_feats, ((0, 0), (6, 0)))
  z = _conv_call(imgflat, gfp, dv_col[:N], kw, kb)
  z = jnp.pad(z, ((0, NP - N), (0, 0)))

  p1 = _make_ax_call(True)(z, src_f, dst_f, ew_f)
  h1 = _mm1_call(p1[0], p1[1], dv_col, W1, b1.reshape(1, 256))

  xcat2 = jnp.concatenate([h1[:, :128], h1[:, 128:]], axis=0)
  p2 = _make_ax_call(False)(xcat2, src_f, dst_f, ew_f)
  w3 = jnp.concatenate([W3a, W3b], axis=1)
  w3 = jnp.pad(w3, ((0, 0), (0, 126)))
  q = _mm2q_call(p2[0], p2[1], dv_col, W2, b2.reshape(1, 512), w3)

  qcat = jnp.concatenate([q[:, 0], q[:, 1]], axis=0)
  p3 = _axq_call()(qcat, src_f, dst_f, ew_f)
  b3 = jnp.concatenate([b3a, b3b]).reshape(2, 1)
  out = _bias_call(p3, dinv, b3)

  mu = out[0, :N].reshape(N, 1)
  log_var = out[1, :N].reshape(N, 1)
  return (mu, log_var)


# speedup vs baseline: 15.0419x; 1.1640x over previous
import functools

import jax
import jax.numpy as jnp
from jax import lax
from jax.experimental import pallas as pl
from jax.experimental.pallas import tpu as pltpu
from jax.experimental.pallas import tpu_sc as plsc

N = 10000
E = 320000
NP = 10240
EF = E + N
K = 320
EP = 348160
NC, NS, L = 2, 16, 16

@functools.cache
def _mesh():
  return plsc.VectorSubcoreMesh(
      core_axis_name="c", subcore_axis_name="s", num_cores=NC, num_subcores=NS)


def _deg_body(dst_hbm, ew_hbm, out_hbm, dst_v, ew_v, acc, sem):
  c = lax.axis_index("c")
  s = lax.axis_index("s")
  w = c * NS + s

  def _z(j, _):
    ew_v[pl.ds(j * L, L)] = jnp.zeros((L,), jnp.float32)
    return 0
  lax.fori_loop(0, K // L, _z, 0)
  pltpu.sync_copy(ew_v, acc.at[pl.ds(s * 640, K)])
  pltpu.sync_copy(ew_v, acc.at[pl.ds(s * 640 + K, K)])
  plsc.subcore_barrier()

  et = EP // (NC * NS)
  nwin = et // K

  def _win(j, _):
    base = w * et + j * K
    pltpu.sync_copy(dst_hbm.at[pl.ds(base, K)], dst_v)
    pltpu.sync_copy(ew_hbm.at[pl.ds(base, K)], ew_v)
    pltpu.sync_copy(ew_v, acc.at[dst_v], add=True)
    return 0
  lax.fori_loop(0, nwin, _win, 0)

  plsc.subcore_barrier()
  pltpu.sync_copy(acc.at[pl.ds(s * 640, 640)],
                  out_hbm.at[c, pl.ds(s * 640, 640)])


@functools.cache
def _deg_call():
  return pl.kernel(
      _deg_body,
      out_type=jax.ShapeDtypeStruct((NC, NP), jnp.float32),
      mesh=_mesh(),
      scratch_types=[
          pltpu.VMEM((K,), jnp.int32),
          pltpu.VMEM((K,), jnp.float32),
          pltpu.VMEM_SHARED((NP,), jnp.float32),
          pltpu.SemaphoreType.DMA,
      ],
  )


def _axq_body(x_hbm, src_hbm, dst_hbm, ew_hbm, out_hbm,
              src_v, dst_v, ew_v, vals_v,
              src2_v, dst2_v, ew2_v, vals2_v, acc, sem, sem2):
  c = lax.axis_index("c")
  s = lax.axis_index("s")
  coff = c * NP

  def _z(j, _):
    ew_v[pl.ds(j * L, L)] = jnp.zeros((L,), jnp.float32)
    return 0
  lax.fori_loop(0, K // L, _z, 0)
  pltpu.sync_copy(ew_v, acc.at[pl.ds(s * 640, K)])
  pltpu.sync_copy(ew_v, acc.at[pl.ds(s * 640 + K, K)])
  plsc.subcore_barrier()

  et = EP // NS
  nwin = et // K

  bufs = ((src_v, dst_v, ew_v, vals_v, sem),
          (src2_v, dst2_v, ew2_v, vals2_v, sem2))

  def _prep(w, b):
    srcb, dstb, ewb, valsb, semb = b
    base = s * et + w * K
    pltpu.sync_copy(src_hbm.at[pl.ds(base, K)], srcb)
    pltpu.sync_copy(dst_hbm.at[pl.ds(base, K)], dstb)
    pltpu.sync_copy(ew_hbm.at[pl.ds(base, K)], ewb)

    def _off(t, _):
      sl = pl.ds(t * L, L)
      srcb[sl] = srcb[sl] + coff
      return 0
    lax.fori_loop(0, K // L, _off, 0)
    pltpu.async_copy(x_hbm.at[srcb], valsb, semb)

  def _proc(b):
    srcb, dstb, ewb, valsb, semb = b
    pltpu.make_async_copy(x_hbm.at[srcb], valsb, semb).wait()

    def _mul(t, _):
      sl = pl.ds(t * L, L)
      valsb[sl] = valsb[sl] * ewb[sl]
      return 0
    lax.fori_loop(0, K // L, _mul, 0)
    pltpu.sync_copy(valsb, acc.at[dstb], add=True)

  _prep(0, bufs[0])

  def _pair(t, _):
    _prep(2 * t + 1, bufs[1])
    _proc(bufs[0])
    _prep(lax.rem(2 * t + 2, nwin), bufs[0])
    _proc(bufs[1])
    return 0
  lax.fori_loop(0, nwin // 2, _pair, 0)
  pltpu.make_async_copy(x_hbm.at[src_v], vals_v, sem).wait()

  plsc.subcore_barrier()
  pltpu.sync_copy(acc.at[pl.ds(s * 640, 640)],
                  out_hbm.at[c, pl.ds(s * 640, 640)])


@functools.cache
def _axq_call():
  return pl.kernel(
      _axq_body,
      out_type=jax.ShapeDtypeStruct((NC, NP), jnp.float32),
      mesh=_mesh(),
      scratch_types=[
          pltpu.VMEM((K,), jnp.int32),
          pltpu.VMEM((K,), jnp.int32),
          pltpu.VMEM((K,), jnp.float32),
          pltpu.VMEM((K,), jnp.float32),
          pltpu.VMEM((K,), jnp.int32),
          pltpu.VMEM((K,), jnp.int32),
          pltpu.VMEM((K,), jnp.float32),
          pltpu.VMEM((K,), jnp.float32),
          pltpu.VMEM_SHARED((NP,), jnp.float32),
          pltpu.SemaphoreType.DMA,
          pltpu.SemaphoreType.DMA,
      ],
  )


DC = 128


def _ax_body(split_edges, x_hbm, src_hbm, dst_hbm, nrm_hbm, out_hbm,
             src_v, dst_v, nrm_v, rows, acc, sem):
  c = lax.axis_index("c")
  s = lax.axis_index("s")

  def _z(r, _):
    for f in range(DC // L):
      rows[r, pl.ds(f * L, L)] = jnp.zeros((L,), jnp.float32)
    return 0
  lax.fori_loop(0, K, _z, 0)
  pltpu.sync_copy(rows, acc.at[pl.ds(s * 640, K)])
  pltpu.sync_copy(rows, acc.at[pl.ds(s * 640 + K, K)])
  plsc.subcore_barrier()

  if split_edges:
    et = EP // (NC * NS)
    first = (c * NS + s) * et
  else:
    et = EP // NS
    first = s * et
  nwin = et // K

  def _win(j, _):
    base = first + j * K
    pltpu.sync_copy(src_hbm.at[pl.ds(base, K)], src_v)
    pltpu.sync_copy(dst_hbm.at[pl.ds(base, K)], dst_v)
    pltpu.sync_copy(nrm_hbm.at[pl.ds(base, K)], nrm_v)

    if not split_edges:
      coff = c * NP

      def _off(t, _):
        sl = pl.ds(t * L, L)
        src_v[sl] = src_v[sl] + coff
        return 0
      lax.fori_loop(0, K // L, _off, 0)

    pltpu.async_copy(x_hbm.at[src_v], rows, sem).wait()

    def _mul(t, _):
      nv = nrm_v[pl.ds(t * L, L)]
      for i in range(L):
        e = t * L + i
        ns = nv[i]
        for f in range(DC // L):
          sl = pl.ds(f * L, L)
          rows[e, sl] = rows[e, sl] * ns
      return 0
    lax.fori_loop(0, K // L, _mul, 0)

    pltpu.sync_copy(rows, acc.at[dst_v], add=True)
    return 0
  lax.fori_loop(0, nwin, _win, 0)

  plsc.subcore_barrier()
  pltpu.sync_copy(acc.at[pl.ds(s * 640, 640)],
                  out_hbm.at[c, pl.ds(s * 640, 640)])


@functools.cache
def _make_ax_call(split_edges):
  return pl.kernel(
      functools.partial(_ax_body, split_edges),
      out_type=jax.ShapeDtypeStruct((NC, NP, DC), jnp.float32),
      mesh=_mesh(),
      scratch_types=[
          pltpu.VMEM((K,), jnp.int32),
          pltpu.VMEM((K,), jnp.int32),
          pltpu.VMEM((K,), jnp.float32),
          pltpu.VMEM((K, DC), jnp.float32),
          pltpu.VMEM_SHARED((NP, DC), jnp.float32),
          pltpu.SemaphoreType.DMA,
      ],
  )


BN_CONV = 200


def _conv_body(img_ref, gfp_ref, dv_ref, kw_ref, kb_ref, out_ref):
  x = img_ref[...]
  col = lax.broadcasted_iota(jnp.int32, (1, 2048), 1)
  wm = col % 32
  hm = (col % 1024) // 32

  zero = jnp.asarray(0.0, jnp.bfloat16)
  shifted = []
  for dy in range(3):
    for dx in range(3):
      o = (dy - 1) * 32 + (dx - 1)
      v = jnp.roll(x, -o, axis=1) if o != 0 else x
      ok = ((hm + (dy - 1) >= 0) & (hm + (dy - 1) < 32)
            & (wm + (dx - 1) >= 0) & (wm + (dx - 1) < 32))
      shifted.append(jnp.where(ok, v, zero))

  feats = []
  for co in range(6):
    acc = jnp.zeros((BN_CONV, 1024), jnp.bfloat16)
    for ci in range(2):
      half = slice(ci * 1024, (ci + 1) * 1024)
      for t in range(9):
        w = kw_ref[co * 18 + ci * 9 + t].astype(jnp.bfloat16)
        acc = acc + shifted[t][:, half] * w
    m = jnp.max(acc, axis=1, keepdims=True).astype(jnp.float32) + kb_ref[co]
    feats.append(jnp.maximum(m, 0.0))
  imgf = jnp.concatenate(feats, axis=1)
  imgf = jnp.pad(imgf, ((0, 0), (0, 122)))
  out_ref[...] = (imgf + gfp_ref[...]) * dv_ref[...]


def _conv_call(imgflat, gfp, dv, kw, kb):
  return pl.pallas_call(
      _conv_body,
      grid=(N // BN_CONV,),
      in_specs=[
          pl.BlockSpec((BN_CONV, 2048), lambda i: (i, 0)),
          pl.BlockSpec((BN_CONV, 128), lambda i: (i, 0)),
          pl.BlockSpec((BN_CONV, 1), lambda i: (i, 0)),
          pl.BlockSpec(memory_space=pltpu.SMEM),
          pl.BlockSpec(memory_space=pltpu.SMEM),
      ],
      out_specs=pl.BlockSpec((BN_CONV, 128), lambda i: (i, 0)),
      out_shape=jax.ShapeDtypeStruct((N, 128), jnp.float32),
  )(imgflat, gfp, dv, kw, kb)


def _dinv_body(degp_ref, out_ref):
  d = jnp.sum(degp_ref[...], axis=0, keepdims=True)
  out_ref[...] = jnp.where(d > 0, lax.rsqrt(d), 0.0)


def _dinv_call(degp):
  return pl.pallas_call(
      _dinv_body,
      out_shape=jax.ShapeDtypeStruct((1, NP), jnp.float32),
  )(degp)


BN_MM = 1024


def _mm1_body(x0_ref, x1_ref, dv_ref, w_ref, b_ref, out_ref):
  dv = dv_ref[...]
  x = (x0_ref[...] + x1_ref[...]) * dv
  h = jnp.dot(x, w_ref[...], preferred_element_type=jnp.float32) + b_ref[...]
  out_ref[...] = jnp.maximum(h, 0.0) * dv


def _mm1_call(x0, x1, dv, w1, b1):
  return pl.pallas_call(
      _mm1_body,
      grid=(NP // BN_MM,),
      in_specs=[
          pl.BlockSpec((BN_MM, 128), lambda i: (i, 0)),
          pl.BlockSpec((BN_MM, 128), lambda i: (i, 0)),
          pl.BlockSpec((BN_MM, 1), lambda i: (i, 0)),
          pl.BlockSpec((128, 256), lambda i: (0, 0)),
          pl.BlockSpec((1, 256), lambda i: (0, 0)),
      ],
      out_specs=pl.BlockSpec((BN_MM, 256), lambda i: (i, 0)),
      out_shape=jax.ShapeDtypeStruct((NP, 256), jnp.float32),
  )(x0, x1, dv, w1, b1)


def _mm2q_body(x0_ref, x1_ref, dv_ref, w_ref, b_ref, w3_ref, out_ref):
  dv = dv_ref[...]
  h = (jnp.dot(x0_ref[...] * dv, w_ref[0:128, :],
               preferred_element_type=jnp.float32)
       + jnp.dot(x1_ref[...] * dv, w_ref[128:256, :],
                 preferred_element_type=jnp.float32)
       + b_ref[...])
  h = jnp.maximum(h, 0.0)
  q = jnp.dot(h, w3_ref[...], preferred_element_type=jnp.float32)
  out_ref[...] = q * dv


def _mm2q_call(x0, x1, dv, w2, b2, w3):
  return pl.pallas_call(
      _mm2q_body,
      grid=(NP // BN_MM,),
      in_specs=[
          pl.BlockSpec((BN_MM, 128), lambda i: (i, 0)),
          pl.BlockSpec((BN_MM, 128), lambda i: (i, 0)),
          pl.BlockSpec((BN_MM, 1), lambda i: (i, 0)),
          pl.BlockSpec((256, 512), lambda i: (0, 0)),
          pl.BlockSpec((1, 512), lambda i: (0, 0)),
          pl.BlockSpec((512, 128), lambda i: (0, 0)),
      ],
      out_specs=pl.BlockSpec((BN_MM, 128), lambda i: (i, 0)),
      out_shape=jax.ShapeDtypeStruct((NP, 128), jnp.float32),
  )(x0, x1, dv, w2, b2, w3)


def _bias_body(p_ref, dv_ref, b_ref, out_ref):
  out_ref[...] = p_ref[...] * dv_ref[...] + b_ref[...]


def _bias_call(p3, dv, b3):
  return pl.pallas_call(
      _bias_body,
      in_specs=[
          pl.BlockSpec((2, NP), lambda: (0, 0)),
          pl.BlockSpec((1, NP), lambda: (0, 0)),
          pl.BlockSpec((2, 1), lambda: (0, 0)),
      ],
      out_specs=pl.BlockSpec((2, NP), lambda: (0, 0)),
      out_shape=jax.ShapeDtypeStruct((2, NP), jnp.float32),
  )(p3, dv, b3)


def kernel(imgbatch, graph_feats, edge_index, edge_weights, conv_k, conv_b,
           W1, b1, W2, b2, W3a, b3a, W3b, b3b):
  src = edge_index[0].astype(jnp.int32)
  dst = edge_index[1].astype(jnp.int32)
  loop = jnp.arange(N, dtype=jnp.int32)
  npad = EP - EF
  pad_idx = jnp.arange(npad, dtype=jnp.int32) % N
  src_f = jnp.concatenate([src, loop, pad_idx])
  dst_f = jnp.concatenate([dst, loop, pad_idx])
  ew_f = jnp.concatenate([edge_weights.astype(jnp.float32),
                          jnp.ones((N,), jnp.float32),
                          jnp.zeros((npad,), jnp.float32)])

  imgflat = imgbatch.reshape(N, 2048).astype(jnp.bfloat16)
  kw = conv_k.reshape(6 * 18)
  kb = conv_b.reshape(6)

  degp = _deg_call()(dst_f, ew_f)
  dinv = _dinv_call(degp)
  dv_col = dinv.reshape(NP, 1)

  gfp = jnp.pad(graph_feats, ((0, 0), (6, 0)))
  z = _conv_call(imgflat, gfp, dv_col[:N], kw, kb)
  z = jnp.pad(z, ((0, NP - N), (0, 0)))

  p1 = _make_ax_call(True)(z, src_f, dst_f, ew_f)
  h1 = _mm1_call(p1[0], p1[1], dv_col, W1, b1.reshape(1, 256))

  xcat2 = jnp.concatenate([h1[:, :128], h1[:, 128:]], axis=0)
  p2 = _make_ax_call(False)(xcat2, src_f, dst_f, ew_f)
  w3 = jnp.concatenate([W3a, W3b], axis=1)
  w3 = jnp.pad(w3, ((0, 0), (0, 126)))
  q = _mm2q_call(p2[0], p2[1], dv_col, W2, b2.reshape(1, 512), w3)

  qcat = jnp.concatenate([q[:, 0], q[:, 1]], axis=0)
  p3 = _axq_call()(qcat, src_f, dst_f, ew_f)
  b3 = jnp.concatenate([b3a, b3b]).reshape(2, 1)
  out = _bias_call(p3, dinv, b3)

  mu = out[0, :N].reshape(N, 1)
  log_var = out[1, :N].reshape(N, 1)
  return (mu, log_var)
